# Initial kernel scaffold; baseline (speedup 1.0000x reference)
import jax
import jax.numpy as jnp
from jax.experimental import pallas as pl


def kernel(x, edge_index, W1, b1, Wh0, bh0, Wh1, bh1, Wf, bf):
    raise NotImplementedError("write your pallas kernel here")


# speedup vs baseline: 8.7453x; 8.7453x over previous
import functools

import jax
import jax.numpy as jnp
from jax import lax
from jax.experimental import pallas as pl
from jax.experimental.pallas import tpu as pltpu
from jax.experimental.pallas import tpu_sc as plsc

N = 10000
E = 320000
D = 128
C = 40

NC = 2
NS = 16
NW = NC * NS

BLK = 128
EPW_BLKS = -(-E // (NW * BLK))
E_PAD = NW * BLK * EPW_BLKS
NROW = 10112
STRIPE = NROW // NS
DW = 128

_mesh = plsc.VectorSubcoreMesh(
    core_axis_name="c", subcore_axis_name="s", num_cores=NC, num_subcores=NS)


def _zero_vmem_rows(ref, nrows, width):
  def row(r, _):
    for j in range(width // 16):
      ref[r, pl.ds(j * 16, 16)] = jnp.zeros((16,), jnp.float32)
    return 0
  lax.fori_loop(0, nrows, row, 0)


def _stripe_copy_zero(zbuf, acc, base, total):
  off = 0
  while off < total:
    ch = min(BLK, total - off)
    pltpu.sync_copy(zbuf.at[pl.ds(0, ch)], acc.at[pl.ds(base + off, ch)])
    off += ch


@functools.partial(
    pl.kernel,
    out_type=jax.ShapeDtypeStruct((NC, NROW, DW), jnp.float32),
    mesh=_mesh,
    scratch_types=[
        pltpu.VMEM((BLK,), jnp.int32),
        pltpu.VMEM((BLK, DW), jnp.float32),
        pltpu.VMEM((BLK, DW), jnp.float32),
        pltpu.VMEM_SHARED((NROW, DW), jnp.float32),
    ],
)
def _sc_deg(dst_hbm, out_hbm, idst, ones_v, zbuf, acc):
  cid = lax.axis_index("c")
  sid = lax.axis_index("s")
  wid = sid * NC + cid

  def fill(r, _):
    for j in range(DW // 16):
      ones_v[r, pl.ds(j * 16, 16)] = jnp.ones((16,), jnp.float32)
      zbuf[r, pl.ds(j * 16, 16)] = jnp.zeros((16,), jnp.float32)
    return 0
  lax.fori_loop(0, BLK, fill, 0)
  _stripe_copy_zero(zbuf, acc, sid * STRIPE, STRIPE)
  plsc.subcore_barrier()

  def body(b, _):
    e0 = (wid * EPW_BLKS + b) * BLK
    pltpu.sync_copy(dst_hbm.at[pl.ds(e0, BLK)], idst)
    pltpu.sync_copy(ones_v, acc.at[idst], add=True)
    return 0
  lax.fori_loop(0, EPW_BLKS, body, 0)
  plsc.subcore_barrier()
  pltpu.sync_copy(acc.at[pl.ds(sid * STRIPE, STRIPE)],
                  out_hbm.at[cid, pl.ds(sid * STRIPE, STRIPE)])


@functools.partial(
    pl.kernel,
    out_type=jax.ShapeDtypeStruct((NC, NROW, D), jnp.float32),
    mesh=_mesh,
    scratch_types=[
        pltpu.VMEM((BLK,), jnp.int32),
        pltpu.VMEM((BLK,), jnp.int32),
        pltpu.VMEM((BLK, D), jnp.float32),
        pltpu.VMEM((BLK, D), jnp.float32),
        pltpu.VMEM_SHARED((NROW, D), jnp.float32),
        pltpu.SemaphoreType.DMA,
    ],
)
def _sc_agg(hp_hbm, src_hbm, dst_hbm, out_hbm, isrc, idst, rows, zbuf, acc, sem):
  cid = lax.axis_index("c")
  sid = lax.axis_index("s")
  wid = sid * NC + cid

  _zero_vmem_rows(zbuf, BLK, D)
  _stripe_copy_zero(zbuf, acc, sid * STRIPE, STRIPE)
  plsc.subcore_barrier()

  def body(b, _):
    e0 = (wid * EPW_BLKS + b) * BLK
    pltpu.sync_copy(src_hbm.at[pl.ds(e0, BLK)], isrc)
    pltpu.async_copy(hp_hbm.at[isrc], rows, sem).wait()
    pltpu.sync_copy(dst_hbm.at[pl.ds(e0, BLK)], idst)
    pltpu.sync_copy(rows, acc.at[idst], add=True)
    return 0
  lax.fori_loop(0, EPW_BLKS, body, 0)
  plsc.subcore_barrier()
  pltpu.sync_copy(acc.at[pl.ds(sid * STRIPE, STRIPE)],
                  out_hbm.at[cid, pl.ds(sid * STRIPE, STRIPE)])


_RB = 1000


def _dinv_body(d0_ref, d1_ref, o_ref):
  deg = d0_ref[:, 0:1] + d1_ref[:, 0:1] + 1.0
  o_ref[...] = jnp.broadcast_to(lax.rsqrt(deg), o_ref.shape)


def _tc_dinv(deg_parts):
  return pl.pallas_call(
      _dinv_body,
      out_shape=jax.ShapeDtypeStruct((NROW, D), jnp.float32),
  )(deg_parts[0], deg_parts[1])


def _mm_scale_body(x_ref, w_ref, dinv_ref, o_ref):
  h = jnp.dot(x_ref[...], w_ref[...], preferred_element_type=jnp.float32)
  o_ref[...] = h * dinv_ref[...]


def _tc_mm_scale(x, w, dinv):
  grid = (N // _RB,)
  return pl.pallas_call(
      _mm_scale_body,
      grid=grid,
      in_specs=[
          pl.BlockSpec((_RB, D), lambda i: (i, 0)),
          pl.BlockSpec((D, D), lambda i: (0, 0)),
          pl.BlockSpec((_RB, D), lambda i: (i, 0)),
      ],
      out_specs=pl.BlockSpec((_RB, D), lambda i: (i, 0)),
      out_shape=jax.ShapeDtypeStruct((N, D), jnp.float32),
  )(x, w, dinv)


def _combine_mm_body(p0_ref, p1_ref, hp_ref, dinv_ref, b_ref, w_ref, o_ref):
  a = dinv_ref[...] * (p0_ref[...] + p1_ref[...] + hp_ref[...]) + b_ref[...]
  a = jnp.maximum(a, 0.0)
  o_ref[...] = jnp.dot(a, w_ref[...], preferred_element_type=jnp.float32)


def _scale_after_body(p0_ref, p1_ref, hp_ref, dinv_ref, b_ref, w_ref, o_ref):
  _combine_mm_body(p0_ref, p1_ref, hp_ref, dinv_ref, b_ref, w_ref, o_ref)
  o_ref[...] = o_ref[...] * dinv_ref[...]


def _final_body(p0_ref, p1_ref, hp_ref, dinv_ref, b_ref, w_ref, bf_ref, o_ref):
  a = dinv_ref[...] * (p0_ref[...] + p1_ref[...] + hp_ref[...]) + b_ref[...]
  a = jnp.maximum(a, 0.0)
  o_ref[...] = jnp.dot(a, w_ref[...], preferred_element_type=jnp.float32) + bf_ref[...]


def _tc_combine(body, parts, hp, dinv, b_row, w, extra=()):
  grid = (N // _RB,)
  blk = pl.BlockSpec((_RB, D), lambda i: (i, 0))
  full = pl.BlockSpec((D, D), lambda i: (0, 0))
  brow = pl.BlockSpec((1, D), lambda i: (0, 0))
  in_specs = [blk, blk, blk, blk, brow, full] + [brow] * len(extra)
  return pl.pallas_call(
      body,
      grid=grid,
      in_specs=in_specs,
      out_specs=blk,
      out_shape=jax.ShapeDtypeStruct((N, D), jnp.float32),
  )(parts[0], parts[1], hp, dinv, b_row, w, *extra)


def kernel(x, edge_index, W1, b1, Wh0, bh0, Wh1, bh1, Wf, bf):
  src = edge_index[0]
  dst = edge_index[1]
  pad = E_PAD - E
  src_p = jnp.concatenate([src, jnp.zeros((pad,), jnp.int32)])
  dst_p = jnp.concatenate([dst, jnp.full((pad,), N, jnp.int32)])

  deg_parts = _sc_deg(dst_p)
  dinv_full = _tc_dinv(deg_parts)
  dinv = dinv_full[:N]

  wf_pad = jnp.zeros((D, D), jnp.float32).at[:, :C].set(Wf)
  bf_pad = jnp.zeros((1, D), jnp.float32).at[0, :C].set(bf)

  hp = _tc_mm_scale(x, W1, dinv)
  parts = _sc_agg(hp, src_p, dst_p)
  hp = _tc_combine(_scale_after_body, (parts[0][:N], parts[1][:N]), hp, dinv,
                   b1.reshape(1, D), Wh0)
  parts = _sc_agg(hp, src_p, dst_p)
  hp = _tc_combine(_scale_after_body, (parts[0][:N], parts[1][:N]), hp, dinv,
                   bh0.reshape(1, D), Wh1)
  parts = _sc_agg(hp, src_p, dst_p)
  out = _tc_combine(_final_body, (parts[0][:N], parts[1][:N]), hp, dinv,
                    bh1.reshape(1, D), wf_pad, extra=(bf_pad,))
  return out[:, :C]
